# baseline (speedup 1.0000x reference)
import jax
import jax.numpy as jnp
from jax import lax
from jax.experimental import pallas as pl
from jax.experimental.pallas import tpu as pltpu

H0, W0, C0 = 46, 78, 3
C1, C2, C3, C4 = 10, 20, 40, 40
W1V = 76
WP1 = 38
W2V = 36
W3V = 34
WP3 = 17
W4V = 15
H1, HP1, H2, H3, HP3, H4 = 44, 22, 20, 18, 9, 7
NACT = 6

L0 = C0 * W0
L1 = C1 * W1V
LP1 = C1 * WP1
L2 = C2 * W2V
L3 = C3 * W3V
LP3 = C3 * WP3
L4 = C4 * W4V


def _bands(wtaps, cin, cout, win, wout):
    wt = wtaps.astype(jnp.bfloat16)
    out = []
    for dy in range(3):
        b4 = jnp.zeros((cin, win, cout, wout), jnp.bfloat16)
        for dx in range(3):
            e = jnp.eye(win, wout, k=-dx, dtype=jnp.bfloat16)
            b4 = b4 + jnp.einsum('ab,ij->aibj', wt[dy * 3 + dx], e)
        out.append(b4.reshape(cin * win, cout * wout))
    return out


def _row_sel(nimg, rin, rout):
    e = jnp.zeros((rout, rin), jnp.bfloat16)
    e = e.at[jnp.arange(rout), 2 * jnp.arange(rout)].set(1.0)
    return jnp.kron(jnp.eye(nimg, dtype=jnp.bfloat16), e)


def _col_sel(cout, win):
    wo = win // 2
    e = jnp.zeros((win, wo), jnp.bfloat16)
    e = e.at[2 * jnp.arange(wo), jnp.arange(wo)].set(1.0)
    return jnp.kron(jnp.eye(cout, dtype=jnp.bfloat16), e)


def _lane_bias(b, width):
    c = b.shape[-1]
    return jnp.repeat(b.reshape(c, 1), width, axis=1).reshape(1, c * width)


def _dot(a, b):
    return jnp.dot(a, b, preferred_element_type=jnp.float32)


def _rollr(x, k):
    if k == 0:
        return x
    return pltpu.roll(x, x.shape[0] - k, 0)


def _fused_kernel(x_ref,
                  c1a, c1b, c1c, c2a, c2b, c2c,
                  c3a, c3b, c3c, c4a, c4b, c4c,
                  b1l, b2l, b3l, b4l,
                  rs1, cs1, rs2, cs2, rs3,
                  wfc1, bfc1, wfc2t, bfc2,
                  out_ref):
    bf = jnp.bfloat16
    x = x_ref[...]

    z = b1l[...] + _dot(x, c1a[...])
    z = z + _dot(_rollr(x, 1), c1b[...])
    z = z + _dot(_rollr(x, 2), c1c[...])
    v = jnp.maximum(z, _rollr(z, 1))
    v = jnp.maximum(v, pltpu.roll(v, v.shape[1] - 1, 1))
    v = jnp.maximum(v, 0.0).astype(bf)
    p1 = _dot(_dot(rs1[...], v).astype(bf), cs1[...]).astype(bf)

    z = b2l[...] + _dot(p1, c2a[...])
    z = z + _dot(_rollr(p1, 1), c2b[...])
    z = z + _dot(_rollr(p1, 2), c2c[...])
    z = jnp.maximum(z, 0.0).astype(bf)

    z3 = b3l[...] + _dot(z, c3a[...])
    z3 = z3 + _dot(_rollr(z, 1), c3b[...])
    z3 = z3 + _dot(_rollr(z, 2), c3c[...])
    v = jnp.maximum(z3, _rollr(z3, 1))
    v = jnp.maximum(v, pltpu.roll(v, v.shape[1] - 1, 1))
    v = jnp.maximum(v, 0.0).astype(bf)
    p3 = _dot(_dot(rs2[...], v).astype(bf), cs2[...]).astype(bf)

    z = b4l[...] + _dot(p3, c4a[...])
    z = z + _dot(_rollr(p3, 1), c4b[...])
    z = z + _dot(_rollr(p3, 2), c4c[...])
    z = jnp.maximum(z, 0.0).astype(bf)

    y = _dot(z, wfc1[...])
    s = y[:, 0:50]
    for h in range(1, H4):
        s = s + _rollr(y[:, h * 50:(h + 1) * 50], h)
    pre = _dot(rs3[...], s.astype(bf)) + bfc1[...]
    h1 = jnp.maximum(pre, 0.0)
    logits = _dot(h1, wfc2t[...]) + bfc2[...]
    m = jnp.max(logits, axis=-1, keepdims=True)
    e = jnp.exp(logits - m)
    out_ref[...] = e / jnp.sum(e, axis=-1, keepdims=True)


def _const_spec(shape):
    return pl.BlockSpec(shape, lambda i: (0,) * len(shape))


def _forward(xr, w1, b1, w2, b2, w3, b3, w4, b4, wfc1, bfc1, wfc2, bfc2):
    g, rows, _ = xr.shape
    nb = rows // H0
    n = g * nb

    c1 = _bands(w1, C0, C1, W0, W1V)
    c2 = _bands(w2, C1, C2, WP1, W2V)
    c3 = _bands(w3, C2, C3, W2V, W3V)
    c4 = _bands(w4, C3, C4, WP3, W4V)
    b1l = _lane_bias(b1, W1V)
    b2l = _lane_bias(b2, W2V)
    b3l = _lane_bias(b3, W3V)
    b4l = _lane_bias(b4, W4V)
    rs1 = _row_sel(nb, H1 + 2, HP1)
    cs1 = _col_sel(C1, W1V)
    rs2 = _row_sel(nb, HP1, HP3)
    cs2 = _col_sel(C3, W3V)
    e3 = jnp.zeros((1, HP3), jnp.bfloat16).at[0, 0].set(1.0)
    rs3 = jnp.kron(jnp.eye(nb, dtype=jnp.bfloat16), e3)
    wf = wfc1.astype(jnp.bfloat16).reshape(50, H4, W4V, C4).transpose(
        3, 2, 1, 0).reshape(C4 * W4V, H4 * 50)
    wfc2t = wfc2.T


    consts = [*c1, *c2, *c3, *c4, b1l, b2l, b3l, b4l,
              rs1, cs1, rs2, cs2, rs3, wf, bfc1, wfc2t, bfc2]
    out = pl.pallas_call(
        _fused_kernel,
        out_shape=jax.ShapeDtypeStruct((g, nb, NACT), jnp.float32),
        grid=(g,),
        in_specs=[pl.BlockSpec((None, nb * H0, L0), lambda i: (i, 0, 0))]
        + [pl.BlockSpec(memory_space=pltpu.MemorySpace.VMEM)
           for _ in consts],
        out_specs=pl.BlockSpec((None, nb, NACT), lambda i: (i, 0, 0)),
        compiler_params=pltpu.CompilerParams(
            dimension_semantics=("parallel",)),
    )(xr, *consts)
    return out.reshape(n, NACT)


def kernel(x, w1, b1, w2, b2, w3, b3, w4, b4, wfc1, bfc1, wfc2, bfc2):
    n = x.shape[0]
    nb = 64 if n % 64 == 0 else (8 if n % 8 == 0 else 1)
    g = n // nb
    xr = jnp.transpose(x.astype(jnp.bfloat16),
                       (0, 2, 1, 3)).reshape(g, nb * H0, L0)
    return _forward(xr, w1, b1, w2, b2, w3, b3, w4, b4,
                    wfc1, bfc1, wfc2, bfc2)


# speedup vs baseline: 1.1198x; 1.1198x over previous
import jax
import jax.numpy as jnp
from jax import lax
from jax.experimental import pallas as pl
from jax.experimental.pallas import tpu as pltpu

H0, W0, C0 = 46, 78, 3
C1, C2, C3, C4 = 10, 20, 40, 40
W1V = 76
WP1 = 38
W2V = 36
W3V = 34
WP3 = 17
W4V = 15
H1, HP1, H2, H3, HP3, H4 = 44, 22, 20, 18, 9, 7
NACT = 6

L0 = C0 * W0
L1 = C1 * W1V
LP1 = C1 * WP1
L2 = C2 * W2V
L3 = C3 * W3V
LP3 = C3 * WP3
L4 = C4 * W4V


def _bands(wtaps, cin, cout, win, wout):
    wt = wtaps.astype(jnp.bfloat16)
    out = []
    for dy in range(3):
        b4 = jnp.zeros((cin, win, cout, wout), jnp.bfloat16)
        for dx in range(3):
            e = jnp.eye(win, wout, k=-dx, dtype=jnp.bfloat16)
            b4 = b4 + jnp.einsum('ab,ij->aibj', wt[dy * 3 + dx], e)
        out.append(b4.reshape(cin * win, cout * wout))
    return out


def _row_sel(nimg, rin, rout):
    e = jnp.zeros((rout, rin), jnp.bfloat16)
    e = e.at[jnp.arange(rout), 2 * jnp.arange(rout)].set(1.0)
    return jnp.kron(jnp.eye(nimg, dtype=jnp.bfloat16), e)


def _col_sel(cout, win):
    wo = win // 2
    e = jnp.zeros((win, wo), jnp.bfloat16)
    e = e.at[2 * jnp.arange(wo), jnp.arange(wo)].set(1.0)
    return jnp.kron(jnp.eye(cout, dtype=jnp.bfloat16), e)


def _lane_bias(b, width):
    c = b.shape[-1]
    return jnp.repeat(b.reshape(c, 1), width, axis=1).reshape(1, c * width)


def _dot(a, b):
    return jnp.dot(a, b, preferred_element_type=jnp.float32)


def _rollr(x, k):
    if k == 0:
        return x
    return pltpu.roll(x, x.shape[0] - k, 0)


def _fused_kernel(x_ref,
                  c1a, c1b, c1c, c2a, c2b, c2c,
                  c3a, c3b, c3c, c4a, c4b, c4c,
                  b1l, b2l, b3l, b4l,
                  rs1, cs1, rs2, cs2, rs3,
                  wfc1, bfc1, wfc2t, bfc2,
                  out_ref):
    bf = jnp.bfloat16
    x = x_ref[...]

    z = b1l[...] + _dot(x, c1a[...])
    z = z + _dot(_rollr(x, 1), c1b[...])
    z = z + _dot(_rollr(x, 2), c1c[...])
    v = jnp.maximum(z, _rollr(z, 1))
    v = jnp.maximum(v, pltpu.roll(v, v.shape[1] - 1, 1))
    v = jnp.maximum(v, 0.0).astype(bf)
    p1 = _dot(_dot(rs1[...], v).astype(bf), cs1[...]).astype(bf)

    z = b2l[...] + _dot(p1, c2a[...])
    z = z + _dot(_rollr(p1, 1), c2b[...])
    z = z + _dot(_rollr(p1, 2), c2c[...])
    z = jnp.maximum(z, 0.0).astype(bf)

    z3 = b3l[...] + _dot(z, c3a[...])
    z3 = z3 + _dot(_rollr(z, 1), c3b[...])
    z3 = z3 + _dot(_rollr(z, 2), c3c[...])
    v = jnp.maximum(z3, _rollr(z3, 1))
    v = jnp.maximum(v, pltpu.roll(v, v.shape[1] - 1, 1))
    v = jnp.maximum(v, 0.0).astype(bf)
    p3 = _dot(_dot(rs2[...], v).astype(bf), cs2[...]).astype(bf)

    z = b4l[...] + _dot(p3, c4a[...])
    z = z + _dot(_rollr(p3, 1), c4b[...])
    z = z + _dot(_rollr(p3, 2), c4c[...])
    z = jnp.maximum(z, 0.0).astype(bf)

    y = _dot(z, wfc1[...])
    s = y[:, 0:50]
    for h in range(1, H4):
        s = s + _rollr(y[:, h * 50:(h + 1) * 50], h)
    pre = _dot(rs3[...], s.astype(bf)) + bfc1[...]
    h1 = jnp.maximum(pre, 0.0)
    logits = _dot(h1, wfc2t[...]) + bfc2[...]
    m = jnp.max(logits, axis=-1, keepdims=True)
    e = jnp.exp(logits - m)
    out_ref[...] = e / jnp.sum(e, axis=-1, keepdims=True)


def _const_spec(shape):
    return pl.BlockSpec(shape, lambda i: (0,) * len(shape))


def _forward(xr, w1, b1, w2, b2, w3, b3, w4, b4, wfc1, bfc1, wfc2, bfc2):
    g, rows, _ = xr.shape
    nb = rows // H0
    n = g * nb

    c1 = _bands(w1, C0, C1, W0, W1V)
    c2 = _bands(w2, C1, C2, WP1, W2V)
    c3 = _bands(w3, C2, C3, W2V, W3V)
    c4 = _bands(w4, C3, C4, WP3, W4V)
    b1l = _lane_bias(b1, W1V)
    b2l = _lane_bias(b2, W2V)
    b3l = _lane_bias(b3, W3V)
    b4l = _lane_bias(b4, W4V)
    rs1 = _row_sel(nb, H1 + 2, HP1)
    cs1 = _col_sel(C1, W1V)
    rs2 = _row_sel(nb, HP1, HP3)
    cs2 = _col_sel(C3, W3V)
    e3 = jnp.zeros((1, HP3), jnp.bfloat16).at[0, 0].set(1.0)
    rs3 = jnp.kron(jnp.eye(nb, dtype=jnp.bfloat16), e3)
    wf = wfc1.astype(jnp.bfloat16).reshape(50, H4, W4V, C4).transpose(
        3, 2, 1, 0).reshape(C4 * W4V, H4 * 50)
    wfc2t = wfc2.T


    consts = [*c1, *c2, *c3, *c4, b1l, b2l, b3l, b4l,
              rs1, cs1, rs2, cs2, rs3, wf, bfc1, wfc2t, bfc2]
    out = pl.pallas_call(
        _fused_kernel,
        out_shape=jax.ShapeDtypeStruct((g, nb, NACT), jnp.float32),
        grid=(g,),
        in_specs=[pl.BlockSpec((None, nb * H0, L0), lambda i: (i, 0, 0))]
        + [pl.BlockSpec(memory_space=pltpu.MemorySpace.VMEM)
           for _ in consts],
        out_specs=pl.BlockSpec((None, nb, NACT), lambda i: (i, 0, 0)),
        compiler_params=pltpu.CompilerParams(
            dimension_semantics=("parallel",)),
    )(xr, *consts)
    return out.reshape(n, NACT)


def kernel(x, w1, b1, w2, b2, w3, b3, w4, b4, wfc1, bfc1, wfc2, bfc2):
    n = x.shape[0]
    nb = 16 if n % 16 == 0 else (8 if n % 8 == 0 else 1)
    g = n // nb
    xr = jnp.transpose(x.astype(jnp.bfloat16),
                       (0, 2, 1, 3)).reshape(g, nb * H0, L0)
    return _forward(xr, w1, b1, w2, b2, w3, b3, w4, b4,
                    wfc1, bfc1, wfc2, bfc2)


# speedup vs baseline: 1.1478x; 1.0251x over previous
import jax
import jax.numpy as jnp
from jax.experimental import pallas as pl
from jax.experimental.pallas import tpu as pltpu

H0, W0, C0 = 46, 78, 3
C1, C2, C3, C4 = 10, 20, 40, 40
W1V = 76
WP1 = 38
W2V = 36
W3V = 34
WP3 = 17
W4V = 15
H1, HP1, H2, H3, HP3, H4 = 44, 22, 20, 18, 9, 7
NACT = 6

L0 = C0 * W0
L1 = C1 * W1V
LP1 = C1 * WP1
L2 = C2 * W2V
L3 = C3 * W3V
LP3 = C3 * WP3
L4 = C4 * W4V


def _bands(wtaps, cin, cout, win, wout):
    wt = wtaps.astype(jnp.bfloat16)
    out = []
    for dy in range(3):
        b4 = jnp.zeros((cin, win, cout, wout), jnp.bfloat16)
        for dx in range(3):
            e = jnp.eye(win, wout, k=-dx, dtype=jnp.bfloat16)
            b4 = b4 + jnp.einsum('ab,ij->aibj', wt[dy * 3 + dx], e)
        out.append(b4.reshape(cin * win, cout * wout))
    return out


def _row_sel(nimg, rin, rout):
    e = jnp.zeros((rout, rin), jnp.bfloat16)
    e = e.at[jnp.arange(rout), 2 * jnp.arange(rout)].set(1.0)
    return jnp.kron(jnp.eye(nimg, dtype=jnp.bfloat16), e)


def _col_sel(cout, win):
    wo = win // 2
    e = jnp.zeros((win, wo), jnp.bfloat16)
    e = e.at[2 * jnp.arange(wo), jnp.arange(wo)].set(1.0)
    return jnp.kron(jnp.eye(cout, dtype=jnp.bfloat16), e)


def _lane_bias(b, width):
    c = b.shape[-1]
    return jnp.repeat(b.reshape(c, 1), width, axis=1).reshape(1, c * width)


def _dot(a, b):
    return jnp.dot(a, b, preferred_element_type=jnp.float32)


def _rollr(x, k):
    if k == 0:
        return x
    return pltpu.roll(x, x.shape[0] - k, 0)


def _fused_kernel(x_ref,
                  c1a, c1b, c1c, c2a, c2b, c2c,
                  c3a, c3b, c3c, c4a, c4b, c4c,
                  b1l, b2l, b3l, b4l,
                  rs1, cs1, rs2, cs2, rs3,
                  wfc1, bfc1, wfc2t, bfc2,
                  out_ref):
    bf = jnp.bfloat16
    x = x_ref[...]

    z = b1l[...] + _dot(x, c1a[...])
    z = z + _dot(_rollr(x, 1), c1b[...])
    z = z + _dot(_rollr(x, 2), c1c[...])
    v = jnp.maximum(z, _rollr(z, 1))
    v = jnp.maximum(v, pltpu.roll(v, v.shape[1] - 1, 1))
    v = jnp.maximum(v, 0.0).astype(bf)
    p1 = _dot(_dot(rs1[...], v).astype(bf), cs1[...]).astype(bf)

    z = b2l[...] + _dot(p1, c2a[...])
    z = z + _dot(_rollr(p1, 1), c2b[...])
    z = z + _dot(_rollr(p1, 2), c2c[...])
    z = jnp.maximum(z, 0.0).astype(bf)

    z3 = b3l[...] + _dot(z, c3a[...])
    z3 = z3 + _dot(_rollr(z, 1), c3b[...])
    z3 = z3 + _dot(_rollr(z, 2), c3c[...])
    v = jnp.maximum(z3, _rollr(z3, 1))
    v = jnp.maximum(v, pltpu.roll(v, v.shape[1] - 1, 1))
    v = jnp.maximum(v, 0.0).astype(bf)
    p3 = _dot(_dot(rs2[...], v).astype(bf), cs2[...]).astype(bf)

    z = b4l[...] + _dot(p3, c4a[...])
    z = z + _dot(_rollr(p3, 1), c4b[...])
    z = z + _dot(_rollr(p3, 2), c4c[...])
    z = jnp.maximum(z, 0.0).astype(bf)

    y = _dot(z, wfc1[...])
    s = y[:, 0:50]
    for h in range(1, H4):
        s = s + _rollr(y[:, h * 50:(h + 1) * 50], h)
    pre = _dot(rs3[...], s.astype(bf)) + bfc1[...]
    h1 = jnp.maximum(pre, 0.0)
    logits = _dot(h1, wfc2t[...]) + bfc2[...]
    m = jnp.max(logits, axis=-1, keepdims=True)
    e = jnp.exp(logits - m)
    out_ref[...] = e / jnp.sum(e, axis=-1, keepdims=True)


def _forward(xr, w1, b1, w2, b2, w3, b3, w4, b4, wfc1, bfc1, wfc2, bfc2):
    g, rows, _ = xr.shape
    nb = rows // H0
    n = g * nb

    c1 = _bands(w1, C0, C1, W0, W1V)
    c2 = _bands(w2, C1, C2, WP1, W2V)
    c3 = _bands(w3, C2, C3, W2V, W3V)
    c4 = _bands(w4, C3, C4, WP3, W4V)
    b1l = _lane_bias(b1, W1V)
    b2l = _lane_bias(b2, W2V)
    b3l = _lane_bias(b3, W3V)
    b4l = _lane_bias(b4, W4V)
    rs1 = _row_sel(nb, H1 + 2, HP1)
    cs1 = _col_sel(C1, W1V)
    rs2 = _row_sel(nb, HP1, HP3)
    cs2 = _col_sel(C3, W3V)
    e3 = jnp.zeros((1, HP3), jnp.bfloat16).at[0, 0].set(1.0)
    rs3 = jnp.kron(jnp.eye(nb, dtype=jnp.bfloat16), e3)
    wf = wfc1.astype(jnp.bfloat16).reshape(50, H4, W4V, C4).transpose(
        3, 2, 1, 0).reshape(C4 * W4V, H4 * 50)
    wfc2t = wfc2.T

    consts = [*c1, *c2, *c3, *c4, b1l, b2l, b3l, b4l,
              rs1, cs1, rs2, cs2, rs3, wf, bfc1, wfc2t, bfc2]
    out = pl.pallas_call(
        _fused_kernel,
        out_shape=jax.ShapeDtypeStruct((g, nb, NACT), jnp.float32),
        grid=(g,),
        in_specs=[pl.BlockSpec((None, nb * H0, L0), lambda i: (i, 0, 0))]
        + [pl.BlockSpec(memory_space=pltpu.MemorySpace.VMEM)
           for _ in consts],
        out_specs=pl.BlockSpec((None, nb, NACT), lambda i: (i, 0, 0)),
        compiler_params=pltpu.CompilerParams(
            dimension_semantics=("parallel",)),
    )(xr, *consts)
    return out.reshape(n, NACT)


def kernel(x, w1, b1, w2, b2, w3, b3, w4, b4, wfc1, bfc1, wfc2, bfc2):
    n = x.shape[0]
    nb = 32 if n % 32 == 0 else (8 if n % 8 == 0 else 1)
    g = n // nb
    xr = jnp.transpose(x.astype(jnp.bfloat16),
                       (0, 2, 1, 3)).reshape(g, nb * H0, L0)
    return _forward(xr, w1, b1, w2, b2, w3, b3, w4, b4,
                    wfc1, bfc1, wfc2, bfc2)
